# Initial kernel scaffold; baseline (speedup 1.0000x reference)
import jax
import jax.numpy as jnp
from jax.experimental import pallas as pl


def kernel(x, a1, b1, a2, b2, w1, c1, w2, c2, w3, c3):
    raise NotImplementedError("write your pallas kernel here")


# speedup vs baseline: 1.1712x; 1.1712x over previous
import jax
import jax.numpy as jnp
from jax.experimental import pallas as pl
from jax.experimental.pallas import tpu as pltpu

_F32 = jnp.float32
_BF16 = jnp.bfloat16


def _body(x_ref, a1p_ref, b1_ref, a2_ref, b2_ref,
          w1_ref, c1_ref, w2_ref, c2_ref, w3_ref, c3_ref,
          out_ref, p1_scr, p2_scr):
    x = x_ref[...]
    b1 = b1_ref[...]
    b2 = b2_ref[...]

    def pool_relu(acc, bias):
        m = jnp.maximum(jnp.maximum(acc[:, 0:128], acc[:, 128:256]),
                        jnp.maximum(acc[:, 256:384], acc[:, 384:512]))
        return jnp.maximum(m + bias, 0.0)

    a1p = a1p_ref[...]
    for p in range(7):
        acc = jnp.dot(x[:, 128 * p:128 * p + 256], a1p,
                      preferred_element_type=_F32)
        for h in range(2):
            r = pool_relu(acc[:, 512 * h:512 * h + 512], b1)
            c0 = 128 * (2 * p + h)
            p1_scr[:, c0:c0 + 128] = r.astype(_BF16)

    a2 = a2_ref[...]
    for yo2 in range(5):
        acc = jnp.dot(p1_scr[:, 256 * yo2:256 * yo2 + 768], a2,
                      preferred_element_type=_F32)
        r = pool_relu(acc, b2)
        p2_scr[:, 128 * yo2:128 * yo2 + 128] = r.astype(_BF16)

    h = jnp.maximum(jnp.dot(p2_scr[...], w1_ref[...],
                            preferred_element_type=_F32) + c1_ref[...], 0.0)
    h = jnp.maximum(jnp.dot(h.astype(_BF16), w2_ref[...],
                            preferred_element_type=_F32) + c2_ref[...], 0.0)
    y = jnp.dot(h.astype(_BF16), w3_ref[...],
                preferred_element_type=_F32) + c3_ref[...]
    out_ref[...] = y.astype(out_ref.dtype)


def kernel(x, a1, b1, a2, b2, w1, c1, w2, c2, w3, c3, *, tb=512):
    B = x.shape[0]
    if B <= tb:
        tb = B
    else:
        tb = max(8, (tb // 8) * 8)
    Bp = pl.cdiv(B, tb) * tb

    xp = jnp.pad(x.reshape(B, 28, 28).astype(_F32),
                 ((0, Bp - B), (2, 2), (2, 2)))
    xf = xp.reshape(Bp, 32 * 32).astype(_BF16)

    a1p = jnp.concatenate([jnp.pad(a1, ((0, 64), (0, 0))),
                           jnp.pad(a1, ((64, 0), (0, 0)))], axis=1)

    weights = (a1p, b1, a2, b2, w1, c1, w2, c2, w3, c3)

    def full(a):
        nd = a.ndim
        return pl.BlockSpec(a.shape, lambda i, _nd=nd: (0,) * _nd)

    out = pl.pallas_call(
        _body,
        out_shape=jax.ShapeDtypeStruct((Bp, 128), _F32),
        grid=(Bp // tb,),
        in_specs=[pl.BlockSpec((tb, 1024), lambda i: (i, 0))] +
                 [full(a) for a in weights],
        out_specs=pl.BlockSpec((tb, 128), lambda i: (i, 0)),
        scratch_shapes=[pltpu.VMEM((tb, 14 * 128), _BF16),
                        pltpu.VMEM((tb, 5 * 128), _BF16)],
        compiler_params=pltpu.CompilerParams(
            dimension_semantics=("parallel",)),
    )(xf, *weights)
    return out[:B, :10]
